# Initial kernel scaffold; baseline (speedup 1.0000x reference)
import jax
import jax.numpy as jnp
from jax.experimental import pallas as pl


def kernel(pos, vel, p_table, particle_type, edge_index):
    raise NotImplementedError("write your pallas kernel here")


# speedup vs baseline: 47.8498x; 47.8498x over previous
import functools

import jax
import jax.numpy as jnp
from jax import lax
from jax.experimental import pallas as pl
from jax.experimental.pallas import tpu as pltpu
from jax.experimental.pallas import tpu_sc as plsc

N_NODES = 100000
N_EDGES = 6400000
NPAD = 100352
ROWS_W = NPAD // 32
ROWS_T = NPAD // 16
CHUNKS = 100
CB = 2048
EPAD = 32 * CHUNKS * CB
A1, A2, A3 = 5e-06, 0.0005, 1e-08

_mesh = plsc.VectorSubcoreMesh(core_axis_name="c", subcore_axis_name="s")
_sc_params = pltpu.CompilerParams(needs_layout_passes=False,
                                  use_tc_tiling_on_sc=False)


@functools.partial(
    pl.kernel,
    out_type=jax.ShapeDtypeStruct((NPAD, 8), jnp.float32),
    mesh=_mesh,
    scratch_types=[
        pltpu.VMEM((ROWS_W, 2), jnp.float32),
        pltpu.VMEM((ROWS_W, 2), jnp.float32),
        pltpu.VMEM((ROWS_W,), jnp.int32),
        pltpu.VMEM((5, 3), jnp.float32),
        pltpu.VMEM((ROWS_W, 8), jnp.float32),
    ],
    compiler_params=_sc_params,
)
def _pack_kernel(pos_h, vel_h, typ_h, ptab_h, out_h, posb, velb, typb, ptab,
                 packb):
    cid = lax.axis_index("c")
    sid = lax.axis_index("s")
    w = sid * 2 + cid
    base = w * ROWS_W
    pltpu.sync_copy(pos_h.at[pl.ds(base, ROWS_W)], posb)
    pltpu.sync_copy(vel_h.at[pl.ds(base, ROWS_W)], velb)
    pltpu.sync_copy(typ_h.at[pl.ds(base, ROWS_W)], typb)
    pltpu.sync_copy(ptab_h, ptab)
    iota = lax.iota(jnp.int32, 16)
    c0 = jnp.zeros((16,), jnp.int32)
    c1 = c0 + 1
    c2 = c0 + 2
    zf = jnp.zeros((16,), jnp.float32)

    def body(g, carry):
        rows = g * 16 + iota
        px = plsc.load_gather(posb, [rows, c0])
        py = plsc.load_gather(posb, [rows, c1])
        vx = plsc.load_gather(velb, [rows, c0])
        vy = plsc.load_gather(velb, [rows, c1])
        t = plsc.load_gather(typb, [rows])
        q0 = plsc.load_gather(ptab, [t, c0])
        q1 = plsc.load_gather(ptab, [t, c1])
        q2 = plsc.load_gather(ptab, [t, c2])
        plsc.store_scatter(packb, [rows, c0], px)
        plsc.store_scatter(packb, [rows, c1], py)
        plsc.store_scatter(packb, [rows, c2], vx)
        plsc.store_scatter(packb, [rows, c0 + 3], vy)
        plsc.store_scatter(packb, [rows, c0 + 4], q0)
        plsc.store_scatter(packb, [rows, c0 + 5], q1)
        plsc.store_scatter(packb, [rows, c0 + 6], q2)
        plsc.store_scatter(packb, [rows, c0 + 7], zf)
        return carry

    lax.fori_loop(0, ROWS_W // 16, body, 0)
    pltpu.sync_copy(packb, out_h.at[pl.ds(base, ROWS_W)])


@functools.partial(
    pl.kernel,
    out_type=jax.ShapeDtypeStruct((2, NPAD, 8), jnp.float32),
    mesh=_mesh,
    scratch_types=[
        pltpu.VMEM((16, 128), jnp.int32),
        pltpu.VMEM((16, 128), jnp.int32),
        pltpu.VMEM((16, 128, 8), jnp.float32),
        pltpu.VMEM((16, 128, 8), jnp.float32),
        pltpu.VMEM((16, 128, 8), jnp.float32),
        pltpu.VMEM_SHARED((NPAD, 8), jnp.float32),
        pltpu.SemaphoreType.DMA,
        pltpu.SemaphoreType.DMA,
    ],
    compiler_params=_sc_params,
)
def _edge_kernel(dst_h, src_h, zeros_h, packed_h, out_h,
                 didx, sidx, nd, ns, msg, acc, sem1, sem2):
    cid = lax.axis_index("c")
    sid = lax.axis_index("s")
    w = sid * 2 + cid
    pltpu.sync_copy(zeros_h.at[pl.ds(sid * ROWS_T, ROWS_T)],
                    acc.at[pl.ds(sid * ROWS_T, ROWS_T)])
    plsc.subcore_barrier()
    iota = lax.iota(jnp.int32, 16)
    c0 = jnp.zeros((16,), jnp.int32)
    c1 = c0 + 1
    zf = jnp.zeros((16,), jnp.float32)

    def zero_body(j, carry):
        jv = jnp.full((16,), j, jnp.int32)
        for g in range(8):
            r = g * 16 + iota
            for cc in range(2, 8):
                plsc.store_scatter(msg, [jv, r, c0 + cc], zf)
        return carry

    lax.fori_loop(0, 16, zero_body, 0)

    def chunk_body(c, carry):
        pltpu.sync_copy(dst_h.at[w, c], didx)
        pltpu.sync_copy(src_h.at[w, c], sidx)
        cps = []
        for j in range(16):
            cps.append(pltpu.async_copy(packed_h.at[didx.at[j]], nd.at[j],
                                        sem1))
            cps.append(pltpu.async_copy(packed_h.at[sidx.at[j]], ns.at[j],
                                        sem2))
        for cp in cps:
            cp.wait()

        def row_body(j, rcarry):
            jv = jnp.full((16,), j, jnp.int32)
            for g in range(8):
                r = g * 16 + iota
                did = plsc.load_gather(didx, [jv, r])
                sidv = plsc.load_gather(sidx, [jv, r])
                pdx = plsc.load_gather(nd, [jv, r, c0])
                pdy = plsc.load_gather(nd, [jv, r, c1])
                vdx = plsc.load_gather(nd, [jv, r, c0 + 2])
                vdy = plsc.load_gather(nd, [jv, r, c0 + 3])
                q0 = plsc.load_gather(nd, [jv, r, c0 + 4])
                q1 = plsc.load_gather(nd, [jv, r, c0 + 5])
                q2 = plsc.load_gather(nd, [jv, r, c0 + 6])
                psx = plsc.load_gather(ns, [jv, r, c0])
                psy = plsc.load_gather(ns, [jv, r, c1])
                vsx = plsc.load_gather(ns, [jv, r, c0 + 2])
                vsy = plsc.load_gather(ns, [jv, r, c0 + 3])
                dpx = psx - pdx
                dpy = psy - pdy
                d2 = dpx * dpx + dpy * dpy
                live = did != sidv
                d2s = jnp.where(live, d2, jnp.float32(1.0))
                t = q2 / d2s
                mf = jnp.where(live, jnp.float32(1.0), jnp.float32(0.0))
                cm = (q0 - t) * mf
                am = q1 * mf
                mx = cm * dpx + am * (vsx - vdx)
                my = cm * dpy + am * (vsy - vdy)
                plsc.store_scatter(msg, [jv, r, c0], mx)
                plsc.store_scatter(msg, [jv, r, c1], my)
            return rcarry

        lax.fori_loop(0, 16, row_body, 0)
        scs = [pltpu.async_copy(msg.at[j], acc.at[didx.at[j]], sem1, add=True)
               for j in range(16)]
        for sc in scs:
            sc.wait()
        return carry

    lax.fori_loop(0, CHUNKS, chunk_body, 0)
    plsc.subcore_barrier()
    pltpu.sync_copy(acc.at[pl.ds(sid * ROWS_T, ROWS_T)],
                    out_h.at[cid, pl.ds(sid * ROWS_T, ROWS_T)])


def _combine_body(a_ref, b_ref, o_ref):
    o_ref[...] = a_ref[...] + b_ref[...]


def kernel(pos, vel, p_table, particle_type, edge_index):
    f32 = jnp.float32
    pos_p = jnp.pad(pos.astype(f32), ((0, NPAD - N_NODES), (0, 0)))
    vel_p = jnp.pad(vel.astype(f32), ((0, NPAD - N_NODES), (0, 0)))
    typ_p = jnp.pad(particle_type.astype(jnp.int32), (0, NPAD - N_NODES))
    ptab = (p_table.astype(f32) * jnp.array([[A1, A2, A3]], f32))
    ei = edge_index.astype(jnp.int32)
    dst4 = jnp.pad(ei[0], (0, EPAD - N_EDGES)).reshape(32, CHUNKS, 16, 128)
    src4 = jnp.pad(ei[1], (0, EPAD - N_EDGES)).reshape(32, CHUNKS, 16, 128)
    zeros = jnp.zeros((NPAD, 8), f32)

    packed = _pack_kernel(pos_p, vel_p, typ_p, ptab)
    partial = _edge_kernel(dst4, src4, zeros, packed)

    a = partial[0].reshape(NPAD * 8 // 256, 256)
    b = partial[1].reshape(NPAD * 8 // 256, 256)
    out = pl.pallas_call(
        _combine_body,
        out_shape=jax.ShapeDtypeStruct(a.shape, f32),
    )(a, b)
    return out.reshape(NPAD, 8)[:N_NODES, :2]


# speedup vs baseline: 57.9775x; 1.2117x over previous
import functools

import jax
import jax.numpy as jnp
from jax import lax
from jax.experimental import pallas as pl
from jax.experimental.pallas import tpu as pltpu
from jax.experimental.pallas import tpu_sc as plsc

N_NODES = 100000
N_EDGES = 6400000
NPAD = 100352
ROWS_W = NPAD // 32
ROWS_T = NPAD // 16
CHUNKS = 200
CB = 1024
CROWS = CB // 128
EPAD = 32 * CHUNKS * CB
A1, A2, A3 = 5e-06, 0.0005, 1e-08

_mesh = plsc.VectorSubcoreMesh(core_axis_name="c", subcore_axis_name="s")
_sc_params = pltpu.CompilerParams(needs_layout_passes=False,
                                  use_tc_tiling_on_sc=False)


@functools.partial(
    pl.kernel,
    out_type=jax.ShapeDtypeStruct((NPAD, 8), jnp.float32),
    mesh=_mesh,
    scratch_types=[
        pltpu.VMEM((ROWS_W, 2), jnp.float32),
        pltpu.VMEM((ROWS_W, 2), jnp.float32),
        pltpu.VMEM((ROWS_W,), jnp.int32),
        pltpu.VMEM((5, 3), jnp.float32),
        pltpu.VMEM((ROWS_W, 8), jnp.float32),
    ],
    compiler_params=_sc_params,
)
def _pack_kernel(pos_h, vel_h, typ_h, ptab_h, out_h, posb, velb, typb, ptab,
                 packb):
    cid = lax.axis_index("c")
    sid = lax.axis_index("s")
    w = sid * 2 + cid
    base = w * ROWS_W
    pltpu.sync_copy(pos_h.at[pl.ds(base, ROWS_W)], posb)
    pltpu.sync_copy(vel_h.at[pl.ds(base, ROWS_W)], velb)
    pltpu.sync_copy(typ_h.at[pl.ds(base, ROWS_W)], typb)
    pltpu.sync_copy(ptab_h, ptab)
    iota = lax.iota(jnp.int32, 16)
    c0 = jnp.zeros((16,), jnp.int32)
    c1 = c0 + 1
    c2 = c0 + 2
    zf = jnp.zeros((16,), jnp.float32)

    def body(g, carry):
        rows = g * 16 + iota
        px = plsc.load_gather(posb, [rows, c0])
        py = plsc.load_gather(posb, [rows, c1])
        vx = plsc.load_gather(velb, [rows, c0])
        vy = plsc.load_gather(velb, [rows, c1])
        t = plsc.load_gather(typb, [rows])
        q0 = plsc.load_gather(ptab, [t, c0])
        q1 = plsc.load_gather(ptab, [t, c1])
        q2 = plsc.load_gather(ptab, [t, c2])
        plsc.store_scatter(packb, [rows, c0], px)
        plsc.store_scatter(packb, [rows, c1], py)
        plsc.store_scatter(packb, [rows, c2], vx)
        plsc.store_scatter(packb, [rows, c0 + 3], vy)
        plsc.store_scatter(packb, [rows, c0 + 4], q0)
        plsc.store_scatter(packb, [rows, c0 + 5], q1)
        plsc.store_scatter(packb, [rows, c0 + 6], q2)
        plsc.store_scatter(packb, [rows, c0 + 7], zf)
        return carry

    lax.fori_loop(0, ROWS_W // 16, body, 0)
    pltpu.sync_copy(packb, out_h.at[pl.ds(base, ROWS_W)])


@functools.partial(
    pl.kernel,
    out_type=jax.ShapeDtypeStruct((2, NPAD, 8), jnp.float32),
    mesh=_mesh,
    scratch_types=[
        [pltpu.VMEM((CROWS, 128), jnp.int32)] * 2,
        [pltpu.VMEM((CROWS, 128), jnp.int32)] * 2,
        [pltpu.VMEM((CROWS, 128, 8), jnp.float32)] * 2,
        [pltpu.VMEM((CROWS, 128, 8), jnp.float32)] * 2,
        pltpu.VMEM((CROWS, 128, 8), jnp.float32),
        pltpu.VMEM_SHARED((NPAD, 8), jnp.float32),
        [pltpu.SemaphoreType.DMA] * 2,
        pltpu.SemaphoreType.DMA,
    ],
    compiler_params=_sc_params,
)
def _edge_kernel(dst_h, src_h, zeros_h, packed_h, out_h,
                 didx2, sidx2, nd2, ns2, msg, acc, semg2, sems):
    cid = lax.axis_index("c")
    sid = lax.axis_index("s")
    w = sid * 2 + cid
    pltpu.sync_copy(zeros_h.at[pl.ds(sid * ROWS_T, ROWS_T)],
                    acc.at[pl.ds(sid * ROWS_T, ROWS_T)])
    plsc.subcore_barrier()
    iota = lax.iota(jnp.int32, 16)
    c0 = jnp.zeros((16,), jnp.int32)
    c1 = c0 + 1
    zf = jnp.zeros((16,), jnp.float32)

    def zero_body(j, carry):
        jv = jnp.full((16,), j, jnp.int32)
        for g in range(8):
            r = g * 16 + iota
            for cc in range(2, 8):
                plsc.store_scatter(msg, [jv, r, c0 + cc], zf)
        return carry

    lax.fori_loop(0, CROWS, zero_body, 0)

    def stage(c, p):
        didx, sidx, nd, ns = didx2[p], sidx2[p], nd2[p], ns2[p]
        pltpu.sync_copy(dst_h.at[w, c], didx)
        pltpu.sync_copy(src_h.at[w, c], sidx)
        for j in range(CROWS):
            pltpu.async_copy(packed_h.at[didx.at[j]], nd.at[j], semg2[p])
            pltpu.async_copy(packed_h.at[sidx.at[j]], ns.at[j], semg2[p])

    def drain(p):
        didx, sidx, nd, ns = didx2[p], sidx2[p], nd2[p], ns2[p]
        for j in range(CROWS):
            pltpu.make_async_copy(packed_h.at[didx.at[j]], nd.at[j],
                                  semg2[p]).wait()
            pltpu.make_async_copy(packed_h.at[sidx.at[j]], ns.at[j],
                                  semg2[p]).wait()

    def compute(p):
        didx, sidx, nd, ns = didx2[p], sidx2[p], nd2[p], ns2[p]

        def row_body(j, rcarry):
            jv = jnp.full((16,), j, jnp.int32)
            for g in range(8):
                r = g * 16 + iota
                did = plsc.load_gather(didx, [jv, r])
                sidv = plsc.load_gather(sidx, [jv, r])
                pdx = plsc.load_gather(nd, [jv, r, c0])
                pdy = plsc.load_gather(nd, [jv, r, c1])
                vdx = plsc.load_gather(nd, [jv, r, c0 + 2])
                vdy = plsc.load_gather(nd, [jv, r, c0 + 3])
                q0 = plsc.load_gather(nd, [jv, r, c0 + 4])
                q1 = plsc.load_gather(nd, [jv, r, c0 + 5])
                q2 = plsc.load_gather(nd, [jv, r, c0 + 6])
                psx = plsc.load_gather(ns, [jv, r, c0])
                psy = plsc.load_gather(ns, [jv, r, c1])
                vsx = plsc.load_gather(ns, [jv, r, c0 + 2])
                vsy = plsc.load_gather(ns, [jv, r, c0 + 3])
                dpx = psx - pdx
                dpy = psy - pdy
                d2 = dpx * dpx + dpy * dpy
                live = did != sidv
                d2s = jnp.where(live, d2, jnp.float32(1.0))
                t = q2 / d2s
                mf = jnp.where(live, jnp.float32(1.0), jnp.float32(0.0))
                cm = (q0 - t) * mf
                am = q1 * mf
                mx = cm * dpx + am * (vsx - vdx)
                my = cm * dpy + am * (vsy - vdy)
                plsc.store_scatter(msg, [jv, r, c0], mx)
                plsc.store_scatter(msg, [jv, r, c1], my)
            return rcarry

        lax.fori_loop(0, CROWS, row_body, 0)

    def scatter(p):
        didx = didx2[p]
        scs = [pltpu.async_copy(msg.at[j], acc.at[didx.at[j]], sems,
                                add=True) for j in range(CROWS)]
        for sc in scs:
            sc.wait()

    stage(0, 0)
    stage(1, 1)

    def pair_body(i, carry):
        c = 2 * i
        for p in range(2):
            drain(p)
            compute(p)
            scatter(p)

            @pl.when(i < CHUNKS // 2 - 1)
            def _():
                stage(c + 2 + p, p)
        return carry

    lax.fori_loop(0, CHUNKS // 2, pair_body, 0)
    plsc.subcore_barrier()
    pltpu.sync_copy(acc.at[pl.ds(sid * ROWS_T, ROWS_T)],
                    out_h.at[cid, pl.ds(sid * ROWS_T, ROWS_T)])


def _combine_body(a_ref, b_ref, o_ref):
    o_ref[...] = a_ref[...] + b_ref[...]


def kernel(pos, vel, p_table, particle_type, edge_index):
    f32 = jnp.float32
    pos_p = jnp.pad(pos.astype(f32), ((0, NPAD - N_NODES), (0, 0)))
    vel_p = jnp.pad(vel.astype(f32), ((0, NPAD - N_NODES), (0, 0)))
    typ_p = jnp.pad(particle_type.astype(jnp.int32), (0, NPAD - N_NODES))
    ptab = (p_table.astype(f32) * jnp.array([[A1, A2, A3]], f32))
    ei = edge_index.astype(jnp.int32)
    dst4 = jnp.pad(ei[0], (0, EPAD - N_EDGES)).reshape(32, CHUNKS, CROWS, 128)
    src4 = jnp.pad(ei[1], (0, EPAD - N_EDGES)).reshape(32, CHUNKS, CROWS, 128)
    zeros = jnp.zeros((NPAD, 8), f32)

    packed = _pack_kernel(pos_p, vel_p, typ_p, ptab)
    partial = _edge_kernel(dst4, src4, zeros, packed)

    a = partial[0].reshape(NPAD * 8 // 256, 256)
    b = partial[1].reshape(NPAD * 8 // 256, 256)
    out = pl.pallas_call(
        _combine_body,
        out_shape=jax.ShapeDtypeStruct(a.shape, f32),
    )(a, b)
    return out.reshape(NPAD, 8)[:N_NODES, :2]


# speedup vs baseline: 58.0182x; 1.0007x over previous
import functools

import jax
import jax.numpy as jnp
from jax import lax
from jax.experimental import pallas as pl
from jax.experimental.pallas import tpu as pltpu
from jax.experimental.pallas import tpu_sc as plsc

N_NODES = 100000
N_EDGES = 6400000
NPAD = 100352
ROWS_W = NPAD // 32
ROWS_T = NPAD // 16
CHUNKS = 200
CB = 1024
CROWS = CB // 128
EPAD = 32 * CHUNKS * CB
A1, A2, A3 = 5e-06, 0.0005, 1e-08

_mesh = plsc.VectorSubcoreMesh(core_axis_name="c", subcore_axis_name="s")
_sc_params = pltpu.CompilerParams(needs_layout_passes=False,
                                  use_tc_tiling_on_sc=False,
                                  skip_device_barrier=True)


@functools.partial(
    pl.kernel,
    out_type=jax.ShapeDtypeStruct((NPAD, 8), jnp.float32),
    mesh=_mesh,
    scratch_types=[
        pltpu.VMEM((ROWS_W, 2), jnp.float32),
        pltpu.VMEM((ROWS_W, 2), jnp.float32),
        pltpu.VMEM((ROWS_W,), jnp.int32),
        pltpu.VMEM((5, 3), jnp.float32),
        pltpu.VMEM((ROWS_W, 8), jnp.float32),
    ],
    compiler_params=_sc_params,
)
def _pack_kernel(pos_h, vel_h, typ_h, ptab_h, out_h, posb, velb, typb, ptab,
                 packb):
    cid = lax.axis_index("c")
    sid = lax.axis_index("s")
    w = sid * 2 + cid
    base = w * ROWS_W
    pltpu.sync_copy(pos_h.at[pl.ds(base, ROWS_W)], posb)
    pltpu.sync_copy(vel_h.at[pl.ds(base, ROWS_W)], velb)
    pltpu.sync_copy(typ_h.at[pl.ds(base, ROWS_W)], typb)
    pltpu.sync_copy(ptab_h, ptab)
    iota = lax.iota(jnp.int32, 16)
    c0 = jnp.zeros((16,), jnp.int32)
    c1 = c0 + 1
    c2 = c0 + 2
    zf = jnp.zeros((16,), jnp.float32)

    def body(g, carry):
        rows = g * 16 + iota
        px = plsc.load_gather(posb, [rows, c0])
        py = plsc.load_gather(posb, [rows, c1])
        vx = plsc.load_gather(velb, [rows, c0])
        vy = plsc.load_gather(velb, [rows, c1])
        t = plsc.load_gather(typb, [rows])
        q0 = plsc.load_gather(ptab, [t, c0])
        q1 = plsc.load_gather(ptab, [t, c1])
        q2 = plsc.load_gather(ptab, [t, c2])
        plsc.store_scatter(packb, [rows, c0], px)
        plsc.store_scatter(packb, [rows, c1], py)
        plsc.store_scatter(packb, [rows, c2], vx)
        plsc.store_scatter(packb, [rows, c0 + 3], vy)
        plsc.store_scatter(packb, [rows, c0 + 4], q0)
        plsc.store_scatter(packb, [rows, c0 + 5], q1)
        plsc.store_scatter(packb, [rows, c0 + 6], q2)
        plsc.store_scatter(packb, [rows, c0 + 7], zf)
        return carry

    lax.fori_loop(0, ROWS_W // 16, body, 0)
    pltpu.sync_copy(packb, out_h.at[pl.ds(base, ROWS_W)])


@functools.partial(
    pl.kernel,
    out_type=jax.ShapeDtypeStruct((2, NPAD, 8), jnp.float32),
    mesh=_mesh,
    scratch_types=[
        [pltpu.VMEM((CROWS, 128), jnp.int32)] * 2,
        [pltpu.VMEM((CROWS, 128), jnp.int32)] * 2,
        [pltpu.VMEM((CROWS, 128, 8), jnp.float32)] * 2,
        [pltpu.VMEM((CROWS, 128, 8), jnp.float32)] * 2,
        pltpu.VMEM((CROWS, 128, 8), jnp.float32),
        pltpu.VMEM_SHARED((NPAD, 8), jnp.float32),
        [pltpu.SemaphoreType.DMA] * 2,
        pltpu.SemaphoreType.DMA,
    ],
    compiler_params=_sc_params,
)
def _edge_kernel(dst_h, src_h, zeros_h, packed_h, out_h,
                 didx2, sidx2, nd2, ns2, msg, acc, semg2, sems):
    cid = lax.axis_index("c")
    sid = lax.axis_index("s")
    w = sid * 2 + cid
    pltpu.sync_copy(zeros_h.at[pl.ds(sid * ROWS_T, ROWS_T)],
                    acc.at[pl.ds(sid * ROWS_T, ROWS_T)])
    plsc.subcore_barrier()
    iota = lax.iota(jnp.int32, 16)
    c0 = jnp.zeros((16,), jnp.int32)
    c1 = c0 + 1
    zf = jnp.zeros((16,), jnp.float32)

    def zero_body(j, carry):
        jv = jnp.full((16,), j, jnp.int32)
        for g in range(8):
            r = g * 16 + iota
            for cc in range(2, 8):
                plsc.store_scatter(msg, [jv, r, c0 + cc], zf)
        return carry

    lax.fori_loop(0, CROWS, zero_body, 0)

    def stage(c, p):
        didx, sidx, nd, ns = didx2[p], sidx2[p], nd2[p], ns2[p]
        pltpu.sync_copy(dst_h.at[w, c], didx)
        pltpu.sync_copy(src_h.at[w, c], sidx)
        for j in range(CROWS):
            pltpu.async_copy(packed_h.at[didx.at[j]], nd.at[j], semg2[p])
            pltpu.async_copy(packed_h.at[sidx.at[j]], ns.at[j], semg2[p])

    def drain(p):
        didx, sidx, nd, ns = didx2[p], sidx2[p], nd2[p], ns2[p]
        for j in range(CROWS):
            pltpu.make_async_copy(packed_h.at[didx.at[j]], nd.at[j],
                                  semg2[p]).wait()
            pltpu.make_async_copy(packed_h.at[sidx.at[j]], ns.at[j],
                                  semg2[p]).wait()

    def compute(p):
        didx, sidx, nd, ns = didx2[p], sidx2[p], nd2[p], ns2[p]

        def row_body(j, rcarry):
            jv = jnp.full((16,), j, jnp.int32)
            for g in range(8):
                r = g * 16 + iota
                did = plsc.load_gather(didx, [jv, r])
                sidv = plsc.load_gather(sidx, [jv, r])
                pdx = plsc.load_gather(nd, [jv, r, c0])
                pdy = plsc.load_gather(nd, [jv, r, c1])
                vdx = plsc.load_gather(nd, [jv, r, c0 + 2])
                vdy = plsc.load_gather(nd, [jv, r, c0 + 3])
                q0 = plsc.load_gather(nd, [jv, r, c0 + 4])
                q1 = plsc.load_gather(nd, [jv, r, c0 + 5])
                q2 = plsc.load_gather(nd, [jv, r, c0 + 6])
                psx = plsc.load_gather(ns, [jv, r, c0])
                psy = plsc.load_gather(ns, [jv, r, c1])
                vsx = plsc.load_gather(ns, [jv, r, c0 + 2])
                vsy = plsc.load_gather(ns, [jv, r, c0 + 3])
                dpx = psx - pdx
                dpy = psy - pdy
                d2 = dpx * dpx + dpy * dpy
                live = did != sidv
                d2s = jnp.where(live, d2, jnp.float32(1.0))
                t = q2 / d2s
                mf = jnp.where(live, jnp.float32(1.0), jnp.float32(0.0))
                cm = (q0 - t) * mf
                am = q1 * mf
                mx = cm * dpx + am * (vsx - vdx)
                my = cm * dpy + am * (vsy - vdy)
                plsc.store_scatter(msg, [jv, r, c0], mx)
                plsc.store_scatter(msg, [jv, r, c1], my)
            return rcarry

        lax.fori_loop(0, CROWS, row_body, 0)

    def scatter(p):
        didx = didx2[p]
        scs = [pltpu.async_copy(msg.at[j], acc.at[didx.at[j]], sems,
                                add=True) for j in range(CROWS)]
        for sc in scs:
            sc.wait()

    stage(0, 0)
    stage(1, 1)

    def pair_body(i, carry):
        c = 2 * i
        for p in range(2):
            drain(p)
            compute(p)
            scatter(p)

            @pl.when(i < CHUNKS // 2 - 1)
            def _():
                stage(c + 2 + p, p)
        return carry

    lax.fori_loop(0, CHUNKS // 2, pair_body, 0)
    plsc.subcore_barrier()
    pltpu.sync_copy(acc.at[pl.ds(sid * ROWS_T, ROWS_T)],
                    out_h.at[cid, pl.ds(sid * ROWS_T, ROWS_T)])


def _combine_body(a_ref, b_ref, o_ref):
    o_ref[...] = a_ref[...] + b_ref[...]


def kernel(pos, vel, p_table, particle_type, edge_index):
    f32 = jnp.float32
    pos_p = jnp.pad(pos.astype(f32), ((0, NPAD - N_NODES), (0, 0)))
    vel_p = jnp.pad(vel.astype(f32), ((0, NPAD - N_NODES), (0, 0)))
    typ_p = jnp.pad(particle_type.astype(jnp.int32), (0, NPAD - N_NODES))
    ptab = (p_table.astype(f32) * jnp.array([[A1, A2, A3]], f32))
    ei = edge_index.astype(jnp.int32)
    dst4 = jnp.pad(ei[0], (0, EPAD - N_EDGES)).reshape(32, CHUNKS, CROWS, 128)
    src4 = jnp.pad(ei[1], (0, EPAD - N_EDGES)).reshape(32, CHUNKS, CROWS, 128)
    zeros = jnp.zeros((NPAD, 8), f32)

    packed = _pack_kernel(pos_p, vel_p, typ_p, ptab)
    partial = _edge_kernel(dst4, src4, zeros, packed)

    a = partial[0].reshape(NPAD * 8 // 256, 256)
    b = partial[1].reshape(NPAD * 8 // 256, 256)
    out = pl.pallas_call(
        _combine_body,
        out_shape=jax.ShapeDtypeStruct(a.shape, f32),
    )(a, b)
    return out.reshape(NPAD, 8)[:N_NODES, :2]


# speedup vs baseline: 58.1561x; 1.0024x over previous
import functools

import jax
import jax.numpy as jnp
from jax import lax
from jax.experimental import pallas as pl
from jax.experimental.pallas import tpu as pltpu
from jax.experimental.pallas import tpu_sc as plsc

N_NODES = 100000
N_EDGES = 6400000
NPAD = 100352
ROWS_W = NPAD // 32
ROWS_T = NPAD // 16
CHUNKS = 200
CB = 1024
CROWS = CB // 128
EPAD = 32 * CHUNKS * CB
A1, A2, A3 = 5e-06, 0.0005, 1e-08

_mesh = plsc.VectorSubcoreMesh(core_axis_name="c", subcore_axis_name="s")
_sc_params = pltpu.CompilerParams(needs_layout_passes=False,
                                  use_tc_tiling_on_sc=False)


@functools.partial(
    pl.kernel,
    out_type=jax.ShapeDtypeStruct((NPAD, 8), jnp.float32),
    mesh=_mesh,
    scratch_types=[
        pltpu.VMEM((ROWS_W, 2), jnp.float32),
        pltpu.VMEM((ROWS_W, 2), jnp.float32),
        pltpu.VMEM((ROWS_W,), jnp.int32),
        pltpu.VMEM((5, 3), jnp.float32),
        pltpu.VMEM((ROWS_W, 8), jnp.float32),
    ],
    compiler_params=_sc_params,
)
def _pack_kernel(pos_h, vel_h, typ_h, ptab_h, out_h, posb, velb, typb, ptab,
                 packb):
    cid = lax.axis_index("c")
    sid = lax.axis_index("s")
    w = sid * 2 + cid
    base = w * ROWS_W
    pltpu.sync_copy(pos_h.at[pl.ds(base, ROWS_W)], posb)
    pltpu.sync_copy(vel_h.at[pl.ds(base, ROWS_W)], velb)
    pltpu.sync_copy(typ_h.at[pl.ds(base, ROWS_W)], typb)
    pltpu.sync_copy(ptab_h, ptab)
    iota = lax.iota(jnp.int32, 16)
    c0 = jnp.zeros((16,), jnp.int32)
    c1 = c0 + 1
    c2 = c0 + 2
    zf = jnp.zeros((16,), jnp.float32)

    def body(g, carry):
        rows = g * 16 + iota
        px = plsc.load_gather(posb, [rows, c0])
        py = plsc.load_gather(posb, [rows, c1])
        vx = plsc.load_gather(velb, [rows, c0])
        vy = plsc.load_gather(velb, [rows, c1])
        t = plsc.load_gather(typb, [rows])
        q0 = plsc.load_gather(ptab, [t, c0])
        q1 = plsc.load_gather(ptab, [t, c1])
        q2 = plsc.load_gather(ptab, [t, c2])
        plsc.store_scatter(packb, [rows, c0], px)
        plsc.store_scatter(packb, [rows, c1], py)
        plsc.store_scatter(packb, [rows, c2], vx)
        plsc.store_scatter(packb, [rows, c0 + 3], vy)
        plsc.store_scatter(packb, [rows, c0 + 4], q0)
        plsc.store_scatter(packb, [rows, c0 + 5], q1)
        plsc.store_scatter(packb, [rows, c0 + 6], q2)
        plsc.store_scatter(packb, [rows, c0 + 7], zf)
        return carry

    lax.fori_loop(0, ROWS_W // 16, body, 0)
    pltpu.sync_copy(packb, out_h.at[pl.ds(base, ROWS_W)])


@functools.partial(
    pl.kernel,
    out_type=jax.ShapeDtypeStruct((2, NPAD, 8), jnp.float32),
    mesh=_mesh,
    scratch_types=[
        [pltpu.VMEM((CB,), jnp.int32)] * 4,
        [pltpu.VMEM((CB,), jnp.int32)] * 4,
        [pltpu.VMEM((CB, 8), jnp.float32)] * 2,
        [pltpu.VMEM((CB, 8), jnp.float32)] * 2,
        pltpu.VMEM((CB, 8), jnp.float32),
        pltpu.VMEM_SHARED((NPAD, 8), jnp.float32),
        [pltpu.SemaphoreType.DMA] * 4,
        [pltpu.SemaphoreType.DMA] * 2,
    ],
    compiler_params=_sc_params,
)
def _edge_kernel(dst_h, src_h, zeros_h, packed_h, out_h,
                 didx4, sidx4, nd2, ns2, msg, acc, semi4, semg2):
    cid = lax.axis_index("c")
    sid = lax.axis_index("s")
    w = sid * 2 + cid
    pltpu.sync_copy(zeros_h.at[pl.ds(sid * ROWS_T, ROWS_T)],
                    acc.at[pl.ds(sid * ROWS_T, ROWS_T)])
    plsc.subcore_barrier()
    iota = lax.iota(jnp.int32, 16)
    c0 = jnp.zeros((16,), jnp.int32)
    c1 = c0 + 1
    zf = jnp.zeros((16,), jnp.float32)

    def zero_body(g, carry):
        rows = g * 16 + iota
        for cc in range(2, 8):
            plsc.store_scatter(msg, [rows, c0 + cc], zf)
        return carry

    lax.fori_loop(0, CB // 16, zero_body, 0)

    def fire_idx(c, q):
        pltpu.async_copy(dst_h.at[w, c], didx4[q], semi4[q])
        pltpu.async_copy(src_h.at[w, c], sidx4[q], semi4[q])

    def wait_idx(q):
        pltpu.make_async_copy(dst_h.at[w, 0], didx4[q], semi4[q]).wait()
        pltpu.make_async_copy(src_h.at[w, 0], sidx4[q], semi4[q]).wait()

    def fire_gather(p, q):
        pltpu.async_copy(packed_h.at[didx4[q]], nd2[p], semg2[p])
        pltpu.async_copy(packed_h.at[sidx4[q]], ns2[p], semg2[p])

    def wait_gather(p, q):
        pltpu.make_async_copy(packed_h.at[didx4[q]], nd2[p], semg2[p]).wait()
        pltpu.make_async_copy(packed_h.at[sidx4[q]], ns2[p], semg2[p]).wait()

    def compute(p, q):
        didx, sidx, nd, ns = didx4[q], sidx4[q], nd2[p], ns2[p]

        def row_body(g, rcarry):
            rows = g * 16 + iota
            did = plsc.load_gather(didx, [rows])
            sidv = plsc.load_gather(sidx, [rows])
            pdx = plsc.load_gather(nd, [rows, c0])
            pdy = plsc.load_gather(nd, [rows, c1])
            vdx = plsc.load_gather(nd, [rows, c0 + 2])
            vdy = plsc.load_gather(nd, [rows, c0 + 3])
            q0 = plsc.load_gather(nd, [rows, c0 + 4])
            q1 = plsc.load_gather(nd, [rows, c0 + 5])
            q2 = plsc.load_gather(nd, [rows, c0 + 6])
            psx = plsc.load_gather(ns, [rows, c0])
            psy = plsc.load_gather(ns, [rows, c1])
            vsx = plsc.load_gather(ns, [rows, c0 + 2])
            vsy = plsc.load_gather(ns, [rows, c0 + 3])
            dpx = psx - pdx
            dpy = psy - pdy
            d2 = dpx * dpx + dpy * dpy
            live = did != sidv
            d2s = jnp.where(live, d2, jnp.float32(1.0))
            t = q2 / d2s
            mf = jnp.where(live, jnp.float32(1.0), jnp.float32(0.0))
            cm = (q0 - t) * mf
            am = q1 * mf
            mx = cm * dpx + am * (vsx - vdx)
            my = cm * dpy + am * (vsy - vdy)
            plsc.store_scatter(msg, [rows, c0], mx)
            plsc.store_scatter(msg, [rows, c1], my)
            return rcarry

        lax.fori_loop(0, CB // 16, row_body, 0)

    def scatter(q):
        pltpu.sync_copy(msg, acc.at[didx4[q]], add=True)

    for q in range(4):
        fire_idx(q, q)
    for k in range(2):
        wait_idx(k)
        fire_gather(k, k)

    def pair_body(i, carry):
        base = 4 * i
        for k in range(4):
            c = base + k
            p = k % 2
            wait_gather(p, k)
            compute(p, k)
            scatter(k)

            @pl.when(c + 4 < CHUNKS)
            def _():
                fire_idx(c + 4, k)

            @pl.when(c + 2 < CHUNKS)
            def _():
                wait_idx((k + 2) % 4)
                fire_gather(p, (k + 2) % 4)
        return carry

    lax.fori_loop(0, CHUNKS // 4, pair_body, 0)
    plsc.subcore_barrier()
    pltpu.sync_copy(acc.at[pl.ds(sid * ROWS_T, ROWS_T)],
                    out_h.at[cid, pl.ds(sid * ROWS_T, ROWS_T)])


def _combine_body(a_ref, b_ref, o_ref):
    o_ref[...] = a_ref[...] + b_ref[...]


def kernel(pos, vel, p_table, particle_type, edge_index):
    f32 = jnp.float32
    pos_p = jnp.pad(pos.astype(f32), ((0, NPAD - N_NODES), (0, 0)))
    vel_p = jnp.pad(vel.astype(f32), ((0, NPAD - N_NODES), (0, 0)))
    typ_p = jnp.pad(particle_type.astype(jnp.int32), (0, NPAD - N_NODES))
    ptab = (p_table.astype(f32) * jnp.array([[A1, A2, A3]], f32))
    ei = edge_index.astype(jnp.int32)
    dst4 = jnp.pad(ei[0], (0, EPAD - N_EDGES)).reshape(32, CHUNKS, CB)
    src4 = jnp.pad(ei[1], (0, EPAD - N_EDGES)).reshape(32, CHUNKS, CB)
    zeros = jnp.zeros((NPAD, 8), f32)

    packed = _pack_kernel(pos_p, vel_p, typ_p, ptab)
    partial = _edge_kernel(dst4, src4, zeros, packed)

    a = partial[0].reshape(NPAD * 8 // 256, 256)
    b = partial[1].reshape(NPAD * 8 // 256, 256)
    out = pl.pallas_call(
        _combine_body,
        out_shape=jax.ShapeDtypeStruct(a.shape, f32),
    )(a, b)
    return out.reshape(NPAD, 8)[:N_NODES, :2]
